# baseline (speedup 1.0000x reference)
import functools

import jax
import jax.numpy as jnp
from jax import lax
from jax.experimental import pallas as pl
from jax.experimental.pallas import tpu as pltpu
from jax.experimental.pallas import tpu_sc as plsc

B, S, D = 4, 8192, 1024
NW = 32
S_PER_W = S // NW
CH = 16
N_CH = S_PER_W // CH
LANES = 16


@functools.partial(
    pl.kernel,
    mesh=plsc.VectorSubcoreMesh(core_axis_name="c", subcore_axis_name="s"),
    out_type=jax.ShapeDtypeStruct((B, S, D), jnp.float32),
    scratch_types=[
        pltpu.VMEM((CH, D), jnp.float32),
        pltpu.VMEM((B, CH, D), jnp.float32),
    ],
)
def _sc_add(x_hbm, pos_hbm, out_hbm, pos_v, x_v):
    wid = lax.axis_index("s") * 2 + lax.axis_index("c")
    base = wid * S_PER_W

    def chunk(ci, carry):
        s0 = base + ci * CH
        pltpu.sync_copy(pos_hbm.at[pl.ds(s0, CH)], pos_v)
        for b in range(B):
            pltpu.sync_copy(x_hbm.at[b, pl.ds(s0, CH)], x_v.at[b])

        def row(i, carry2):
            for j in range(D // LANES):
                sl = pl.ds(j * LANES, LANES)
                pv = pos_v[i, sl]
                for b in range(B):
                    x_v[b, i, sl] = x_v[b, i, sl] + pv
            return carry2

        lax.fori_loop(0, CH, row, 0)
        for b in range(B):
            pltpu.sync_copy(x_v.at[b], out_hbm.at[b, pl.ds(s0, CH)])
        return carry

    lax.fori_loop(0, N_CH, chunk, 0)


def kernel(x, pos_embed):
    return _sc_add(x, pos_embed)


# speedup vs baseline: 1.5698x; 1.5698x over previous
import functools

import jax
import jax.numpy as jnp
from jax import lax
from jax.experimental import pallas as pl
from jax.experimental.pallas import tpu as pltpu
from jax.experimental.pallas import tpu_sc as plsc

B, S, D = 4, 8192, 1024
NW = 32
S_PER_W = S // NW
CH = 16
N_CH = S_PER_W // CH
LANES = 16
SC_BATCH = 3


@functools.partial(
    pl.kernel,
    mesh=plsc.VectorSubcoreMesh(core_axis_name="c", subcore_axis_name="s"),
    out_type=jax.ShapeDtypeStruct((1, S, D), jnp.float32),
    scratch_types=[
        pltpu.VMEM((CH, D), jnp.float32),
        pltpu.VMEM((CH, D), jnp.float32),
    ],
)
def _sc_add(x_hbm, pos_hbm, out_hbm, pos_v, x_v):
    wid = lax.axis_index("s") * 2 + lax.axis_index("c")
    base = wid * S_PER_W

    def chunk(ci, carry):
        s0 = base + ci * CH
        pltpu.sync_copy(pos_hbm.at[pl.ds(s0, CH)], pos_v)
        pltpu.sync_copy(x_hbm.at[SC_BATCH, pl.ds(s0, CH)], x_v)

        def row(i, carry2):
            for j in range(D // LANES):
                sl = pl.ds(j * LANES, LANES)
                x_v[i, sl] = x_v[i, sl] + pos_v[i, sl]
            return carry2

        lax.fori_loop(0, CH, row, 0)
        pltpu.sync_copy(x_v, out_hbm.at[0, pl.ds(s0, CH)])
        return carry

    lax.fori_loop(0, N_CH, chunk, 0)


def _add_body(x_ref, p_ref, o_ref):
    o_ref[...] = x_ref[...] + p_ref[...]


def kernel(x, pos_embed):
    BLK = 2048
    n_s = S // BLK
    tc_out = pl.pallas_call(
        _add_body,
        grid=(n_s, B - 1),
        in_specs=[
            pl.BlockSpec((1, BLK, D), lambda s, b: (b, s, 0)),
            pl.BlockSpec((BLK, D), lambda s, b: (s, 0)),
        ],
        out_specs=pl.BlockSpec((1, BLK, D), lambda s, b: (b, s, 0)),
        out_shape=jax.ShapeDtypeStruct((B - 1, S, D), x.dtype),
    )(x, pos_embed)
    sc_out = _sc_add(x, pos_embed)
    return jnp.concatenate([tc_out, sc_out], axis=0)


# speedup vs baseline: 2.0679x; 1.3173x over previous
import functools

import jax
import jax.numpy as jnp
from jax import lax
from jax.experimental import pallas as pl
from jax.experimental.pallas import tpu as pltpu
from jax.experimental.pallas import tpu_sc as plsc

B, S, D = 4, 8192, 1024
NW = 32
S_PER_W = S // NW
CH = 8
N_CH = S_PER_W // CH
LANES = 16


@functools.partial(
    pl.kernel,
    mesh=plsc.VectorSubcoreMesh(core_axis_name="c", subcore_axis_name="s"),
    out_type=jax.ShapeDtypeStruct((B, S, D), jnp.float32),
    scratch_types=[
        pltpu.VMEM((2, CH, D), jnp.float32),
        pltpu.VMEM((2, B, CH, D), jnp.float32),
        pltpu.SemaphoreType.DMA,
        pltpu.SemaphoreType.DMA,
    ],
)
def _sc_add(x_hbm, pos_hbm, out_hbm, pos_v, x_v, sem_in, sem_out):
    wid = lax.axis_index("s") * 2 + lax.axis_index("c")
    base = wid * S_PER_W

    def fire_in(ci, p):
        s0 = base + ci * CH
        pltpu.async_copy(pos_hbm.at[pl.ds(s0, CH)], pos_v.at[p], sem_in)
        pltpu.async_copy(x_hbm.at[:, pl.ds(s0, CH)], x_v.at[p], sem_in)

    def drain_in(p):
        pltpu.make_async_copy(
            pos_hbm.at[pl.ds(base, CH)], pos_v.at[p], sem_in).wait()
        pltpu.make_async_copy(
            x_hbm.at[:, pl.ds(base, CH)], x_v.at[p], sem_in).wait()

    def fire_out(ci, p):
        s0 = base + ci * CH
        pltpu.async_copy(x_v.at[p], out_hbm.at[:, pl.ds(s0, CH)], sem_out)

    def drain_out(p):
        pltpu.make_async_copy(
            x_v.at[p], out_hbm.at[:, pl.ds(base, CH)], sem_out).wait()

    fire_in(0, 0)

    def body(ci, carry):
        p = lax.rem(ci, 2)
        q = 1 - p
        drain_in(p)

        @pl.when(ci >= 1)
        def _():
            drain_out(q)

        @pl.when(ci + 1 < N_CH)
        def _():
            fire_in(ci + 1, q)

        def row(i, carry2):
            for j in range(D // LANES):
                sl = pl.ds(j * LANES, LANES)
                pv = pos_v[p, i, sl]
                for b in range(B):
                    x_v[p, b, i, sl] = x_v[p, b, i, sl] + pv
            return carry2

        lax.fori_loop(0, CH, row, 0)
        fire_out(ci, p)
        return carry

    lax.fori_loop(0, N_CH, body, 0)
    drain_out(lax.rem(N_CH - 1, 2))


def kernel(x, pos_embed):
    return _sc_add(x, pos_embed)


# speedup vs baseline: 3.7433x; 1.8102x over previous
import jax
import jax.numpy as jnp
from jax.experimental import pallas as pl
from jax.experimental.pallas import tpu as pltpu


def _add_body(x_ref, p_ref, o_ref):
    o_ref[...] = x_ref[...] + p_ref[...]


def kernel(x, pos_embed):
    B, S, D = x.shape
    BLK = 2048
    n_s = S // BLK
    x2 = x.reshape(B * S, D)
    out = pl.pallas_call(
        _add_body,
        grid=(n_s, B),
        in_specs=[
            pl.BlockSpec((BLK, D), lambda s, b: (b * n_s + s, 0)),
            pl.BlockSpec((BLK, D), lambda s, b: (s, 0)),
        ],
        out_specs=pl.BlockSpec((BLK, D), lambda s, b: (b * n_s + s, 0)),
        out_shape=jax.ShapeDtypeStruct((B * S, D), x.dtype),
        compiler_params=pltpu.CompilerParams(vmem_limit_bytes=120 * 1024 * 1024),
    )(x2, pos_embed)
    return out.reshape(B, S, D)
